# baseline (speedup 1.0000x reference)
import numpy as np
import jax
import jax.numpy as jnp
from jax.experimental import pallas as pl
from jax.experimental.pallas import tpu as pltpu

B = 16
C = 3
H = 224
W = 224
P = 16
NP = (H // P) * (W // P)
NT = NP + 1
NPR = H // P
PATCH_DIM = C * P * P
HIDDEN = 768
K = 8192
THRESH = 0.85

M = B * NP
KT = 512
KSTEPS = K // KT
F32_MIN = float(jnp.finfo(jnp.float32).min)
I32_MAX = np.int32(2**31 - 1)

_lanes = np.arange(PATCH_DIM)
_px, _ch, _py = _lanes // 48, (_lanes % 48) // 16, _lanes % 16
_PERM = (_ch * 256 + _py * 16 + _px).astype(np.int32)
_PSIG = np.zeros((PATCH_DIM, PATCH_DIM), np.float32)
_PSIG[np.arange(PATCH_DIM), _PERM] = 1.0


def _transpose_kernel(pv_ref, t_ref):
    for pr in range(NPR):
        s = pv_ref[0, :, pl.ds(pr * P, P), :]
        s2 = jnp.concatenate([s[c] for c in range(C)], axis=0)
        t_ref[0, pr] = jnp.swapaxes(s2, 0, 1)


def _patch_kernel(x_ref, psig_ref, w_ref, b_ref, pe_ref, pn_ref, emb_ref):
    x = x_ref[0]
    n = jnp.sqrt(jnp.sum(x * x, axis=1, keepdims=True))
    pn = x / jnp.maximum(n, 1e-12)
    pn_ref[0] = jax.lax.dot_general(pn, psig_ref[...], (((1,), (0,)), ((), ())),
                                    preferred_element_type=jnp.float32)
    e = jax.lax.dot_general(x, w_ref[...], (((1,), (0,)), ((), ())),
                            preferred_element_type=jnp.float32)
    emb_ref[0] = e + b_ref[0][None, :] + pe_ref[...]


def _dist_kernel(pn_ref, v_ref, min_ref, lab_ref, accs_ref, acci_ref):
    k = pl.program_id(0)

    @pl.when(k == 0)
    def _init():
        accs_ref[...] = jnp.full((M, KT), jnp.inf, jnp.float32)
        acci_ref[...] = jnp.zeros((M, KT), jnp.int32)

    v = v_ref[...]
    n2 = jax.lax.dot_general(v * v, jnp.ones((PATCH_DIM, 1), jnp.float32),
                             (((1,), (0,)), ((), ())),
                             preferred_element_type=jnp.float32)
    vn = v / jnp.maximum(jnp.sqrt(n2), 1e-12)
    d = jax.lax.dot_general(pn_ref[...], vn, (((1,), (1,)), ((), ())),
                            preferred_element_type=jnp.float32)
    s = 1.0 - d
    better = s < accs_ref[...]
    accs_ref[...] = jnp.where(better, s, accs_ref[...])
    acci_ref[...] = jnp.where(better, k, acci_ref[...])

    @pl.when(k == KSTEPS - 1)
    def _fin():
        val = accs_ref[...]
        m = jnp.min(val, axis=1, keepdims=True)
        gid = acci_ref[...] * KT + jax.lax.broadcasted_iota(
            jnp.int32, (M, KT), 1)
        sel = jnp.where(val == m, gid, I32_MAX)
        min_ref[...] = m
        lab_ref[...] = jnp.min(sel, axis=1)[:, None]


MB = 4


def _merge_kernel(lab_ref, ms_ref, e_ref, cls_ref, pos0_ref, pad_ref,
                  batch_ref, labout_ref, attn_ref):
    for g in range(MB):
        _merge_one(g, lab_ref, ms_ref, e_ref, cls_ref, pos0_ref, pad_ref,
                   batch_ref, labout_ref, attn_ref)


def _merge_one(g, lab_ref, ms_ref, e_ref, cls_ref, pos0_ref, pad_ref,
               batch_ref, labout_ref, attn_ref):
    lab = lab_ref[g, 0]
    ms = ms_ref[g, 0]

    pos = jax.lax.broadcasted_iota(jnp.int32, (NP, NP), 1)
    ent = jax.lax.broadcasted_iota(jnp.int32, (NP, NP), 0)
    tri = pos < ent

    msk = ms > THRESH
    unm = ~msk

    eq = lab[None, :] == lab[:, None]
    lt = lab[None, :] < lab[:, None]

    had_earlier = jnp.sum((eq & unm[None, :] & tri).astype(jnp.int32), axis=1)
    first = unm & (had_earlier == 0)

    distinct_lt = jnp.sum((first[None, :] & lt).astype(jnp.int32), axis=1)
    n_distinct = jnp.sum(first.astype(jnp.int32))
    masked_before = jnp.sum((msk[None, :] & tri).astype(jnp.int32), axis=1)

    final = jnp.where(msk, n_distinct + masked_before, distinct_lt)
    labout_ref[g, 0] = final

    t = final + 1
    lrow = jax.lax.broadcasted_iota(jnp.int32, (NT, NP), 0)
    oh = (t[None, :] == lrow).astype(jnp.float32)
    sums = jax.lax.dot_general(oh, e_ref[g], (((1,), (0,)), ((), ())),
                               preferred_element_type=jnp.float32)
    counts = jnp.sum(oh, axis=1)[:, None]
    mean = sums / jnp.maximum(counts, 1.0)
    rows = jnp.where(counts > 0.0, mean, pad_ref[0])
    batch_ref[g] = rows
    batch_ref[g, 0:1, :] = cls_ref[0] + pos0_ref[0]

    li = jax.lax.broadcasted_iota(jnp.int32, (1, NT), 1)[0]
    bm = (counts[:, 0] == 0.0) & (li >= 1)
    attn_ref[g, 0] = jnp.broadcast_to(
        jnp.where(bm, F32_MIN, 0.0)[None, :], (NT, NT))


def kernel(pixel_values, vocab, W_patch, b_patch, cls_token, pos_embed, pad_token):
    pe_body = pos_embed[0, 1:, :]
    pos0 = pos_embed[:, 0:1, :]
    b2 = b_patch.reshape(1, HIDDEN)
    psig = jnp.asarray(_PSIG)
    w_perm = W_patch[jnp.asarray(_PERM), :]

    t = pl.pallas_call(
        _transpose_kernel,
        grid=(B,),
        in_specs=[pl.BlockSpec((1, C, H, W), lambda i: (i, 0, 0, 0))],
        out_specs=pl.BlockSpec((1, NPR, W, C * P), lambda i: (i, 0, 0, 0)),
        out_shape=jax.ShapeDtypeStruct((B, NPR, W, C * P), jnp.float32),
        compiler_params=pltpu.CompilerParams(
            dimension_semantics=("arbitrary",)),
    )(pixel_values)
    x_unperm = t.reshape(B, NP, PATCH_DIM)

    pn, emb = pl.pallas_call(
        _patch_kernel,
        grid=(B,),
        in_specs=[
            pl.BlockSpec((1, NP, PATCH_DIM), lambda i: (i, 0, 0)),
            pl.BlockSpec((PATCH_DIM, PATCH_DIM), lambda i: (0, 0)),
            pl.BlockSpec((PATCH_DIM, HIDDEN), lambda i: (0, 0)),
            pl.BlockSpec((1, HIDDEN), lambda i: (0, 0)),
            pl.BlockSpec((NP, HIDDEN), lambda i: (0, 0)),
        ],
        out_specs=[
            pl.BlockSpec((1, NP, PATCH_DIM), lambda i: (i, 0, 0)),
            pl.BlockSpec((1, NP, HIDDEN), lambda i: (i, 0, 0)),
        ],
        out_shape=[
            jax.ShapeDtypeStruct((B, NP, PATCH_DIM), jnp.float32),
            jax.ShapeDtypeStruct((B, NP, HIDDEN), jnp.float32),
        ],
        compiler_params=pltpu.CompilerParams(
            dimension_semantics=("arbitrary",)),
    )(x_unperm, psig, w_perm, b2, pe_body)

    min_s, labels = pl.pallas_call(
        _dist_kernel,
        grid=(KSTEPS,),
        in_specs=[
            pl.BlockSpec((M, PATCH_DIM), lambda k: (0, 0)),
            pl.BlockSpec((KT, PATCH_DIM), lambda k: (k, 0)),
        ],
        out_specs=[
            pl.BlockSpec((M, 1), lambda k: (0, 0)),
            pl.BlockSpec((M, 1), lambda k: (0, 0)),
        ],
        out_shape=[
            jax.ShapeDtypeStruct((M, 1), jnp.float32),
            jax.ShapeDtypeStruct((M, 1), jnp.int32),
        ],
        scratch_shapes=[pltpu.VMEM((M, KT), jnp.float32),
                        pltpu.VMEM((M, KT), jnp.int32)],
        compiler_params=pltpu.CompilerParams(
            dimension_semantics=("arbitrary",)),
    )(pn.reshape(M, PATCH_DIM), vocab)

    lab_r = labels.reshape(B, 1, NP)
    ms_r = min_s.reshape(B, 1, NP)

    batch, labout, attn = pl.pallas_call(
        _merge_kernel,
        grid=(B // MB,),
        in_specs=[
            pl.BlockSpec((MB, 1, NP), lambda i: (i, 0, 0)),
            pl.BlockSpec((MB, 1, NP), lambda i: (i, 0, 0)),
            pl.BlockSpec((MB, NP, HIDDEN), lambda i: (i, 0, 0)),
            pl.BlockSpec((1, 1, HIDDEN), lambda i: (0, 0, 0)),
            pl.BlockSpec((1, 1, HIDDEN), lambda i: (0, 0, 0)),
            pl.BlockSpec((1, 1, HIDDEN), lambda i: (0, 0, 0)),
        ],
        out_specs=[
            pl.BlockSpec((MB, NT, HIDDEN), lambda i: (i, 0, 0)),
            pl.BlockSpec((MB, 1, NP), lambda i: (i, 0, 0)),
            pl.BlockSpec((MB, 1, NT, NT), lambda i: (i, 0, 0, 0)),
        ],
        out_shape=[
            jax.ShapeDtypeStruct((B, NT, HIDDEN), jnp.float32),
            jax.ShapeDtypeStruct((B, 1, NP), jnp.int32),
            jax.ShapeDtypeStruct((B, 1, NT, NT), jnp.float32),
        ],
        compiler_params=pltpu.CompilerParams(
            dimension_semantics=("arbitrary",)),
    )(lab_r, ms_r, emb, cls_token, pos0, pad_token)

    return batch, labout.reshape(B, NP), attn


# speedup vs baseline: 1.0374x; 1.0374x over previous
import numpy as np
import jax
import jax.numpy as jnp
from jax.experimental import pallas as pl
from jax.experimental.pallas import tpu as pltpu

B = 16
C = 3
H = 224
W = 224
P = 16
NP = (H // P) * (W // P)
NT = NP + 1
NPR = H // P
PATCH_DIM = C * P * P
HIDDEN = 768
K = 8192
THRESH = 0.85

M = B * NP
KT = 512
KSTEPS = K // KT
F32_MIN = float(jnp.finfo(jnp.float32).min)
I32_MAX = np.int32(2**31 - 1)

_lanes = np.arange(PATCH_DIM)
_px, _ch, _py = _lanes // 48, (_lanes % 48) // 16, _lanes % 16
_PERM = (_ch * 256 + _py * 16 + _px).astype(np.int32)
_PSIG = np.zeros((PATCH_DIM, PATCH_DIM), np.float32)
_PSIG[np.arange(PATCH_DIM), _PERM] = 1.0


def _transpose_kernel(pv_ref, t_ref):
    for pr in range(NPR):
        s = pv_ref[0, :, pl.ds(pr * P, P), :]
        s2 = jnp.concatenate([s[c] for c in range(C)], axis=0)
        t_ref[0, pr] = jnp.swapaxes(s2, 0, 1)


def _patch_kernel(x_ref, psig_ref, w_ref, b_ref, pe_ref, pn_ref, emb_ref):
    x = x_ref[0]
    n = jnp.sqrt(jnp.sum(x * x, axis=1, keepdims=True))
    pn = x / jnp.maximum(n, 1e-12)
    pn_ref[0] = jax.lax.dot_general(pn, psig_ref[...], (((1,), (0,)), ((), ())),
                                    preferred_element_type=jnp.float32)
    e = jax.lax.dot_general(x, w_ref[...], (((1,), (0,)), ((), ())),
                            preferred_element_type=jnp.float32)
    emb_ref[0] = e + b_ref[0][None, :] + pe_ref[...]


def _dist_kernel(pn_ref, v_ref, min_ref, lab_ref, accs_ref, acci_ref):
    k = pl.program_id(0)

    @pl.when(k == 0)
    def _init():
        accs_ref[...] = jnp.full((M, KT), jnp.inf, jnp.float32)
        acci_ref[...] = jnp.zeros((M, KT), jnp.int32)

    v = v_ref[...]
    vn = v / jnp.maximum(jnp.sqrt(jnp.sum(v * v, axis=1, keepdims=True)), 1e-12)
    d = jax.lax.dot_general(pn_ref[...], vn, (((1,), (1,)), ((), ())),
                            preferred_element_type=jnp.float32)
    s = 1.0 - d
    better = s < accs_ref[...]
    accs_ref[...] = jnp.where(better, s, accs_ref[...])
    acci_ref[...] = jnp.where(better, k, acci_ref[...])

    @pl.when(k == KSTEPS - 1)
    def _fin():
        val = accs_ref[...]
        m = jnp.min(val, axis=1, keepdims=True)
        gid = acci_ref[...] * KT + jax.lax.broadcasted_iota(
            jnp.int32, (M, KT), 1)
        sel = jnp.where(val == m, gid, I32_MAX)
        min_ref[...] = m
        lab_ref[...] = jnp.min(sel, axis=1)[:, None]


MB = 4


def _merge_kernel(lab_ref, ms_ref, e_ref, cls_ref, pos0_ref, pad_ref,
                  batch_ref, labout_ref, attn_ref):
    for g in range(MB):
        _merge_one(g, lab_ref, ms_ref, e_ref, cls_ref, pos0_ref, pad_ref,
                   batch_ref, labout_ref, attn_ref)


def _merge_one(g, lab_ref, ms_ref, e_ref, cls_ref, pos0_ref, pad_ref,
               batch_ref, labout_ref, attn_ref):
    lab = lab_ref[g, 0]
    ms = ms_ref[g, 0]

    pos = jax.lax.broadcasted_iota(jnp.int32, (NP, NP), 1)
    ent = jax.lax.broadcasted_iota(jnp.int32, (NP, NP), 0)
    tri = pos < ent

    msk = ms > THRESH
    unm = ~msk

    eq = lab[None, :] == lab[:, None]
    lt = lab[None, :] < lab[:, None]

    had_earlier = jnp.sum((eq & unm[None, :] & tri).astype(jnp.int32), axis=1)
    first = unm & (had_earlier == 0)

    distinct_lt = jnp.sum((first[None, :] & lt).astype(jnp.int32), axis=1)
    n_distinct = jnp.sum(first.astype(jnp.int32))
    masked_before = jnp.sum((msk[None, :] & tri).astype(jnp.int32), axis=1)

    final = jnp.where(msk, n_distinct + masked_before, distinct_lt)
    labout_ref[g, 0] = final

    t = final + 1
    lrow = jax.lax.broadcasted_iota(jnp.int32, (NT, NP), 0)
    oh = (t[None, :] == lrow).astype(jnp.float32)
    sums = jax.lax.dot_general(oh, e_ref[g], (((1,), (0,)), ((), ())),
                               preferred_element_type=jnp.float32)
    counts = jnp.sum(oh, axis=1)[:, None]
    mean = sums / jnp.maximum(counts, 1.0)
    rows = jnp.where(counts > 0.0, mean, pad_ref[0])
    batch_ref[g] = rows
    batch_ref[g, 0:1, :] = cls_ref[0] + pos0_ref[0]

    li = jax.lax.broadcasted_iota(jnp.int32, (1, NT), 1)[0]
    bm = (counts[:, 0] == 0.0) & (li >= 1)
    attn_ref[g, 0] = jnp.broadcast_to(
        jnp.where(bm, F32_MIN, 0.0)[None, :], (NT, NT))


def kernel(pixel_values, vocab, W_patch, b_patch, cls_token, pos_embed, pad_token):
    pe_body = pos_embed[0, 1:, :]
    pos0 = pos_embed[:, 0:1, :]
    b2 = b_patch.reshape(1, HIDDEN)
    psig = jnp.asarray(_PSIG)
    w_perm = W_patch[jnp.asarray(_PERM), :]

    t = pl.pallas_call(
        _transpose_kernel,
        grid=(B,),
        in_specs=[pl.BlockSpec((1, C, H, W), lambda i: (i, 0, 0, 0))],
        out_specs=pl.BlockSpec((1, NPR, W, C * P), lambda i: (i, 0, 0, 0)),
        out_shape=jax.ShapeDtypeStruct((B, NPR, W, C * P), jnp.float32),
        compiler_params=pltpu.CompilerParams(
            dimension_semantics=("arbitrary",)),
    )(pixel_values)
    x_unperm = t.reshape(B, NP, PATCH_DIM)

    pn, emb = pl.pallas_call(
        _patch_kernel,
        grid=(B,),
        in_specs=[
            pl.BlockSpec((1, NP, PATCH_DIM), lambda i: (i, 0, 0)),
            pl.BlockSpec((PATCH_DIM, PATCH_DIM), lambda i: (0, 0)),
            pl.BlockSpec((PATCH_DIM, HIDDEN), lambda i: (0, 0)),
            pl.BlockSpec((1, HIDDEN), lambda i: (0, 0)),
            pl.BlockSpec((NP, HIDDEN), lambda i: (0, 0)),
        ],
        out_specs=[
            pl.BlockSpec((1, NP, PATCH_DIM), lambda i: (i, 0, 0)),
            pl.BlockSpec((1, NP, HIDDEN), lambda i: (i, 0, 0)),
        ],
        out_shape=[
            jax.ShapeDtypeStruct((B, NP, PATCH_DIM), jnp.float32),
            jax.ShapeDtypeStruct((B, NP, HIDDEN), jnp.float32),
        ],
        compiler_params=pltpu.CompilerParams(
            dimension_semantics=("arbitrary",)),
    )(x_unperm, psig, w_perm, b2, pe_body)

    min_s, labels = pl.pallas_call(
        _dist_kernel,
        grid=(KSTEPS,),
        in_specs=[
            pl.BlockSpec((M, PATCH_DIM), lambda k: (0, 0)),
            pl.BlockSpec((KT, PATCH_DIM), lambda k: (k, 0)),
        ],
        out_specs=[
            pl.BlockSpec((M, 1), lambda k: (0, 0)),
            pl.BlockSpec((M, 1), lambda k: (0, 0)),
        ],
        out_shape=[
            jax.ShapeDtypeStruct((M, 1), jnp.float32),
            jax.ShapeDtypeStruct((M, 1), jnp.int32),
        ],
        scratch_shapes=[pltpu.VMEM((M, KT), jnp.float32),
                        pltpu.VMEM((M, KT), jnp.int32)],
        compiler_params=pltpu.CompilerParams(
            dimension_semantics=("arbitrary",)),
    )(pn.reshape(M, PATCH_DIM), vocab)

    lab_r = labels.reshape(B, 1, NP)
    ms_r = min_s.reshape(B, 1, NP)

    batch, labout, attn = pl.pallas_call(
        _merge_kernel,
        grid=(B // MB,),
        in_specs=[
            pl.BlockSpec((MB, 1, NP), lambda i: (i, 0, 0)),
            pl.BlockSpec((MB, 1, NP), lambda i: (i, 0, 0)),
            pl.BlockSpec((MB, NP, HIDDEN), lambda i: (i, 0, 0)),
            pl.BlockSpec((1, 1, HIDDEN), lambda i: (0, 0, 0)),
            pl.BlockSpec((1, 1, HIDDEN), lambda i: (0, 0, 0)),
            pl.BlockSpec((1, 1, HIDDEN), lambda i: (0, 0, 0)),
        ],
        out_specs=[
            pl.BlockSpec((MB, NT, HIDDEN), lambda i: (i, 0, 0)),
            pl.BlockSpec((MB, 1, NP), lambda i: (i, 0, 0)),
            pl.BlockSpec((MB, 1, NT, NT), lambda i: (i, 0, 0, 0)),
        ],
        out_shape=[
            jax.ShapeDtypeStruct((B, NT, HIDDEN), jnp.float32),
            jax.ShapeDtypeStruct((B, 1, NP), jnp.int32),
            jax.ShapeDtypeStruct((B, 1, NT, NT), jnp.float32),
        ],
        compiler_params=pltpu.CompilerParams(
            dimension_semantics=("arbitrary",)),
    )(lab_r, ms_r, emb, cls_token, pos0, pad_token)

    return batch, labout.reshape(B, NP), attn


# speedup vs baseline: 1.1097x; 1.0697x over previous
import numpy as np
import jax
import jax.numpy as jnp
from jax.experimental import pallas as pl
from jax.experimental.pallas import tpu as pltpu

B = 16
C = 3
H = 224
W = 224
P = 16
NP = (H // P) * (W // P)
NT = NP + 1
NPR = H // P
PATCH_DIM = C * P * P
HIDDEN = 768
K = 8192
THRESH = 0.85

M = B * NP
KT = 256
KSTEPS = K // KT
F32_MIN = float(jnp.finfo(jnp.float32).min)
I32_MAX = np.int32(2**31 - 1)

_lanes = np.arange(PATCH_DIM)
_px, _ch, _py = _lanes // 48, (_lanes % 48) // 16, _lanes % 16
_PERM = (_ch * 256 + _py * 16 + _px).astype(np.int32)
_PSIG = np.zeros((PATCH_DIM, PATCH_DIM), np.float32)
_PSIG[np.arange(PATCH_DIM), _PERM] = 1.0


TB = 4


def _transpose_kernel(pv_ref, t_ref):
    for g in range(TB):
        for pr in range(NPR):
            s = pv_ref[g, :, pl.ds(pr * P, P), :]
            s2 = jnp.concatenate([s[c] for c in range(C)], axis=0)
            t_ref[g, pr] = jnp.swapaxes(s2, 0, 1)


def _dist_kernel(x_ref, v_ref, psig_ref, w_ref, b_ref, min_ref, lab_ref,
                 emb_ref, pn_ref, accs_ref, acci_ref):
    k = pl.program_id(0)

    @pl.when(k == 0)
    def _init():
        x = x_ref[...]
        n = jnp.sqrt(jnp.sum(x * x, axis=1, keepdims=True))
        pnu = x / jnp.maximum(n, 1e-12)
        pn_ref[...] = jax.lax.dot_general(
            pnu, psig_ref[...], (((1,), (0,)), ((), ())),
            preferred_element_type=jnp.float32)
        emb_ref[...] = jax.lax.dot_general(
            x, w_ref[...], (((1,), (0,)), ((), ())),
            preferred_element_type=jnp.float32) + b_ref[0][None, :]
        accs_ref[...] = jnp.full((M, KT), jnp.inf, jnp.float32)
        acci_ref[...] = jnp.zeros((M, KT), jnp.int32)

    v = v_ref[...]
    vn = v / jnp.maximum(jnp.sqrt(jnp.sum(v * v, axis=1, keepdims=True)), 1e-12)
    d = jax.lax.dot_general(pn_ref[...], vn, (((1,), (1,)), ((), ())),
                            preferred_element_type=jnp.float32)
    s = 1.0 - d
    better = s < accs_ref[...]
    accs_ref[...] = jnp.where(better, s, accs_ref[...])
    acci_ref[...] = jnp.where(better, k, acci_ref[...])

    @pl.when(k == KSTEPS - 1)
    def _fin():
        val = accs_ref[...]
        m = jnp.min(val, axis=1, keepdims=True)
        gid = acci_ref[...] * KT + jax.lax.broadcasted_iota(
            jnp.int32, (M, KT), 1)
        sel = jnp.where(val == m, gid, I32_MAX)
        min_ref[...] = m
        lab_ref[...] = jnp.min(sel, axis=1)[:, None]


MB = 16


def _merge_kernel(lab_ref, ms_ref, e_ref, pe_ref, cls_ref, pos0_ref, pad_ref,
                  batch_ref, labout_ref, attn_ref):
    for g in range(MB):
        _merge_one(g, lab_ref, ms_ref, e_ref, pe_ref, cls_ref, pos0_ref,
                   pad_ref, batch_ref, labout_ref, attn_ref)


def _merge_one(g, lab_ref, ms_ref, e_ref, pe_ref, cls_ref, pos0_ref, pad_ref,
               batch_ref, labout_ref, attn_ref):
    lab = lab_ref[g, 0]
    ms = ms_ref[g, 0]

    pos = jax.lax.broadcasted_iota(jnp.int32, (NP, NP), 1)
    ent = jax.lax.broadcasted_iota(jnp.int32, (NP, NP), 0)
    tri = pos < ent

    msk = ms > THRESH
    unm = ~msk

    eq = lab[None, :] == lab[:, None]
    lt = lab[None, :] < lab[:, None]

    had_earlier = jnp.sum((eq & unm[None, :] & tri).astype(jnp.int32), axis=1)
    first = unm & (had_earlier == 0)

    distinct_lt = jnp.sum((first[None, :] & lt).astype(jnp.int32), axis=1)
    n_distinct = jnp.sum(first.astype(jnp.int32))
    masked_before = jnp.sum((msk[None, :] & tri).astype(jnp.int32), axis=1)

    final = jnp.where(msk, n_distinct + masked_before, distinct_lt)
    labout_ref[g, 0] = final

    t = final + 1
    lrow = jax.lax.broadcasted_iota(jnp.int32, (NT, NP), 0)
    oh = (t[None, :] == lrow).astype(jnp.float32)
    e = e_ref[g] + pe_ref[...]
    sums = jax.lax.dot_general(oh, e, (((1,), (0,)), ((), ())),
                               preferred_element_type=jnp.float32)
    counts = jnp.sum(oh, axis=1)[:, None]
    mean = sums / jnp.maximum(counts, 1.0)
    rows = jnp.where(counts > 0.0, mean, pad_ref[0])
    batch_ref[g] = rows
    batch_ref[g, 0:1, :] = cls_ref[0] + pos0_ref[0]

    li = jax.lax.broadcasted_iota(jnp.int32, (1, NT), 1)[0]
    bm = (counts[:, 0] == 0.0) & (li >= 1)
    attn_ref[g, 0] = jnp.broadcast_to(
        jnp.where(bm, F32_MIN, 0.0)[None, :], (NT, NT))


def kernel(pixel_values, vocab, W_patch, b_patch, cls_token, pos_embed, pad_token):
    pe_body = pos_embed[0, 1:, :]
    pos0 = pos_embed[:, 0:1, :]
    b2 = b_patch.reshape(1, HIDDEN)
    psig = jnp.asarray(_PSIG)
    w_perm = W_patch[jnp.asarray(_PERM), :]

    t = pl.pallas_call(
        _transpose_kernel,
        grid=(B // TB,),
        in_specs=[pl.BlockSpec((TB, C, H, W), lambda i: (i, 0, 0, 0))],
        out_specs=pl.BlockSpec((TB, NPR, W, C * P), lambda i: (i, 0, 0, 0)),
        out_shape=jax.ShapeDtypeStruct((B, NPR, W, C * P), jnp.float32),
        compiler_params=pltpu.CompilerParams(
            dimension_semantics=("arbitrary",)),
    )(pixel_values)
    x_unperm = t.reshape(B, NP, PATCH_DIM)

    min_s, labels, emb = pl.pallas_call(
        _dist_kernel,
        grid=(KSTEPS,),
        in_specs=[
            pl.BlockSpec((M, PATCH_DIM), lambda k: (0, 0)),
            pl.BlockSpec((KT, PATCH_DIM), lambda k: (k, 0)),
            pl.BlockSpec((PATCH_DIM, PATCH_DIM), lambda k: (0, 0)),
            pl.BlockSpec((PATCH_DIM, HIDDEN), lambda k: (0, 0)),
            pl.BlockSpec((1, HIDDEN), lambda k: (0, 0)),
        ],
        out_specs=[
            pl.BlockSpec((M, 1), lambda k: (0, 0)),
            pl.BlockSpec((M, 1), lambda k: (0, 0)),
            pl.BlockSpec((M, HIDDEN), lambda k: (0, 0)),
        ],
        out_shape=[
            jax.ShapeDtypeStruct((M, 1), jnp.float32),
            jax.ShapeDtypeStruct((M, 1), jnp.int32),
            jax.ShapeDtypeStruct((M, HIDDEN), jnp.float32),
        ],
        scratch_shapes=[pltpu.VMEM((M, PATCH_DIM), jnp.float32),
                        pltpu.VMEM((M, KT), jnp.float32),
                        pltpu.VMEM((M, KT), jnp.int32)],
        compiler_params=pltpu.CompilerParams(
            dimension_semantics=("arbitrary",)),
    )(x_unperm.reshape(M, PATCH_DIM), vocab, psig, w_perm, b2)

    lab_r = labels.reshape(B, 1, NP)
    ms_r = min_s.reshape(B, 1, NP)
    emb_r = emb.reshape(B, NP, HIDDEN)

    batch, labout, attn = pl.pallas_call(
        _merge_kernel,
        grid=(B // MB,),
        in_specs=[
            pl.BlockSpec((MB, 1, NP), lambda i: (i, 0, 0)),
            pl.BlockSpec((MB, 1, NP), lambda i: (i, 0, 0)),
            pl.BlockSpec((MB, NP, HIDDEN), lambda i: (i, 0, 0)),
            pl.BlockSpec((NP, HIDDEN), lambda i: (0, 0)),
            pl.BlockSpec((1, 1, HIDDEN), lambda i: (0, 0, 0)),
            pl.BlockSpec((1, 1, HIDDEN), lambda i: (0, 0, 0)),
            pl.BlockSpec((1, 1, HIDDEN), lambda i: (0, 0, 0)),
        ],
        out_specs=[
            pl.BlockSpec((MB, NT, HIDDEN), lambda i: (i, 0, 0)),
            pl.BlockSpec((MB, 1, NP), lambda i: (i, 0, 0)),
            pl.BlockSpec((MB, 1, NT, NT), lambda i: (i, 0, 0, 0)),
        ],
        out_shape=[
            jax.ShapeDtypeStruct((B, NT, HIDDEN), jnp.float32),
            jax.ShapeDtypeStruct((B, 1, NP), jnp.int32),
            jax.ShapeDtypeStruct((B, 1, NT, NT), jnp.float32),
        ],
        compiler_params=pltpu.CompilerParams(
            dimension_semantics=("arbitrary",)),
    )(lab_r, ms_r, emb_r, pe_body, cls_token, pos0, pad_token)

    return batch, labout.reshape(B, NP), attn


# speedup vs baseline: 1.1398x; 1.0271x over previous
import numpy as np
import jax
import jax.numpy as jnp
from jax.experimental import pallas as pl
from jax.experimental.pallas import tpu as pltpu

B = 16
C = 3
H = 224
W = 224
P = 16
NP = (H // P) * (W // P)
NT = NP + 1
NPR = H // P
PATCH_DIM = C * P * P
HIDDEN = 768
K = 8192
THRESH = 0.85

M = B * NP
KT = 256
KSTEPS = K // KT
F32_MIN = float(jnp.finfo(jnp.float32).min)
I32_MAX = np.int32(2**31 - 1)

_lanes = np.arange(PATCH_DIM)
_px, _ch, _py = _lanes // 48, (_lanes % 48) // 16, _lanes % 16
_PERM = (_ch * 256 + _py * 16 + _px).astype(np.int32)
_PSIG = np.zeros((PATCH_DIM, PATCH_DIM), np.float32)
_PSIG[np.arange(PATCH_DIM), _PERM] = 1.0


TB = 4


def _transpose_kernel(pv_ref, t_ref):
    for g in range(TB):
        for pr in range(NPR):
            s = pv_ref[g, :, pl.ds(pr * P, P), :]
            s2 = jnp.concatenate([s[c] for c in range(C)], axis=0)
            t_ref[g, pr] = jnp.swapaxes(s2, 0, 1)


def _dist_kernel(x_ref, v_ref, psig_ref, w_ref, b_ref, min_ref, lab_ref,
                 emb_ref, pn_ref, accs_ref, acci_ref):
    k = pl.program_id(0)

    @pl.when(k == 0)
    def _init():
        x = x_ref[...]
        n = jnp.sqrt(jnp.sum(x * x, axis=1, keepdims=True))
        pnu = x / jnp.maximum(n, 1e-12)
        pn_ref[...] = jax.lax.dot_general(
            pnu, psig_ref[...], (((1,), (0,)), ((), ())),
            preferred_element_type=jnp.float32)
        emb_ref[...] = jax.lax.dot_general(
            x, w_ref[...], (((1,), (0,)), ((), ())),
            preferred_element_type=jnp.float32) + b_ref[0][None, :]
        accs_ref[...] = jnp.full((M, KT), jnp.inf, jnp.float32)
        acci_ref[...] = jnp.zeros((M, KT), jnp.int32)

    v = v_ref[...]
    vn = v / jnp.maximum(jnp.sqrt(jnp.sum(v * v, axis=1, keepdims=True)), 1e-12)
    d = jax.lax.dot_general(pn_ref[...], vn, (((1,), (1,)), ((), ())),
                            preferred_element_type=jnp.float32)
    s = 1.0 - d
    better = s < accs_ref[...]
    accs_ref[...] = jnp.where(better, s, accs_ref[...])
    acci_ref[...] = jnp.where(better, k, acci_ref[...])

    @pl.when(k == KSTEPS - 1)
    def _fin():
        val = accs_ref[...]
        m = jnp.min(val, axis=1, keepdims=True)
        gid = acci_ref[...] * KT + jax.lax.broadcasted_iota(
            jnp.int32, (M, KT), 1)
        sel = jnp.where(val == m, gid, I32_MAX)
        min_ref[...] = m
        lab_ref[...] = jnp.min(sel, axis=1)[:, None]


MB = 8


def _merge_kernel(lab_ref, ms_ref, e_ref, pe_ref, cls_ref, pos0_ref, pad_ref,
                  batch_ref, labout_ref, attn_ref):
    for g in range(MB):
        _merge_one(g, lab_ref, ms_ref, e_ref, pe_ref, cls_ref, pos0_ref,
                   pad_ref, batch_ref, labout_ref, attn_ref)


def _merge_one(g, lab_ref, ms_ref, e_ref, pe_ref, cls_ref, pos0_ref, pad_ref,
               batch_ref, labout_ref, attn_ref):
    lab = lab_ref[g, 0]
    ms = ms_ref[g, 0]

    pos = jax.lax.broadcasted_iota(jnp.int32, (NP, NP), 1)
    ent = jax.lax.broadcasted_iota(jnp.int32, (NP, NP), 0)
    tri = pos < ent

    msk = ms > THRESH
    unm = ~msk

    eq = lab[None, :] == lab[:, None]
    lt = lab[None, :] < lab[:, None]

    had_earlier = jnp.sum((eq & unm[None, :] & tri).astype(jnp.int32), axis=1)
    first = unm & (had_earlier == 0)

    distinct_lt = jnp.sum((first[None, :] & lt).astype(jnp.int32), axis=1)
    n_distinct = jnp.sum(first.astype(jnp.int32))
    masked_before = jnp.sum((msk[None, :] & tri).astype(jnp.int32), axis=1)

    final = jnp.where(msk, n_distinct + masked_before, distinct_lt)
    labout_ref[g, 0] = final

    t = final + 1
    lrow = jax.lax.broadcasted_iota(jnp.int32, (NT, NP), 0)
    oh = (t[None, :] == lrow).astype(jnp.float32)
    e = e_ref[g] + pe_ref[...]
    sums = jax.lax.dot_general(oh, e, (((1,), (0,)), ((), ())),
                               preferred_element_type=jnp.float32)
    counts = jnp.sum(oh, axis=1)[:, None]
    mean = sums / jnp.maximum(counts, 1.0)
    rows = jnp.where(counts > 0.0, mean, pad_ref[0])
    batch_ref[g] = rows
    batch_ref[g, 0:1, :] = cls_ref[0] + pos0_ref[0]

    li = jax.lax.broadcasted_iota(jnp.int32, (1, NT), 1)[0]
    bm = (counts[:, 0] == 0.0) & (li >= 1)
    attn_ref[g, 0] = jnp.broadcast_to(
        jnp.where(bm, F32_MIN, 0.0)[None, :], (NT, NT))


def kernel(pixel_values, vocab, W_patch, b_patch, cls_token, pos_embed, pad_token):
    pe_body = pos_embed[0, 1:, :]
    pos0 = pos_embed[:, 0:1, :]
    b2 = b_patch.reshape(1, HIDDEN)
    psig = jnp.asarray(_PSIG)
    w_perm = W_patch[jnp.asarray(_PERM), :]

    t = pl.pallas_call(
        _transpose_kernel,
        grid=(B // TB,),
        in_specs=[pl.BlockSpec((TB, C, H, W), lambda i: (i, 0, 0, 0))],
        out_specs=pl.BlockSpec((TB, NPR, W, C * P), lambda i: (i, 0, 0, 0)),
        out_shape=jax.ShapeDtypeStruct((B, NPR, W, C * P), jnp.float32),
        compiler_params=pltpu.CompilerParams(
            dimension_semantics=("arbitrary",)),
    )(pixel_values)
    x_unperm = t.reshape(B, NP, PATCH_DIM)

    min_s, labels, emb = pl.pallas_call(
        _dist_kernel,
        grid=(KSTEPS,),
        in_specs=[
            pl.BlockSpec((M, PATCH_DIM), lambda k: (0, 0)),
            pl.BlockSpec((KT, PATCH_DIM), lambda k: (k, 0)),
            pl.BlockSpec((PATCH_DIM, PATCH_DIM), lambda k: (0, 0)),
            pl.BlockSpec((PATCH_DIM, HIDDEN), lambda k: (0, 0)),
            pl.BlockSpec((1, HIDDEN), lambda k: (0, 0)),
        ],
        out_specs=[
            pl.BlockSpec((M, 1), lambda k: (0, 0)),
            pl.BlockSpec((M, 1), lambda k: (0, 0)),
            pl.BlockSpec((M, HIDDEN), lambda k: (0, 0)),
        ],
        out_shape=[
            jax.ShapeDtypeStruct((M, 1), jnp.float32),
            jax.ShapeDtypeStruct((M, 1), jnp.int32),
            jax.ShapeDtypeStruct((M, HIDDEN), jnp.float32),
        ],
        scratch_shapes=[pltpu.VMEM((M, PATCH_DIM), jnp.float32),
                        pltpu.VMEM((M, KT), jnp.float32),
                        pltpu.VMEM((M, KT), jnp.int32)],
        compiler_params=pltpu.CompilerParams(
            dimension_semantics=("arbitrary",)),
    )(x_unperm.reshape(M, PATCH_DIM), vocab, psig, w_perm, b2)

    lab_r = labels.reshape(B, 1, NP)
    ms_r = min_s.reshape(B, 1, NP)
    emb_r = emb.reshape(B, NP, HIDDEN)

    batch, labout, attn = pl.pallas_call(
        _merge_kernel,
        grid=(B // MB,),
        in_specs=[
            pl.BlockSpec((MB, 1, NP), lambda i: (i, 0, 0)),
            pl.BlockSpec((MB, 1, NP), lambda i: (i, 0, 0)),
            pl.BlockSpec((MB, NP, HIDDEN), lambda i: (i, 0, 0)),
            pl.BlockSpec((NP, HIDDEN), lambda i: (0, 0)),
            pl.BlockSpec((1, 1, HIDDEN), lambda i: (0, 0, 0)),
            pl.BlockSpec((1, 1, HIDDEN), lambda i: (0, 0, 0)),
            pl.BlockSpec((1, 1, HIDDEN), lambda i: (0, 0, 0)),
        ],
        out_specs=[
            pl.BlockSpec((MB, NT, HIDDEN), lambda i: (i, 0, 0)),
            pl.BlockSpec((MB, 1, NP), lambda i: (i, 0, 0)),
            pl.BlockSpec((MB, 1, NT, NT), lambda i: (i, 0, 0, 0)),
        ],
        out_shape=[
            jax.ShapeDtypeStruct((B, NT, HIDDEN), jnp.float32),
            jax.ShapeDtypeStruct((B, 1, NP), jnp.int32),
            jax.ShapeDtypeStruct((B, 1, NT, NT), jnp.float32),
        ],
        compiler_params=pltpu.CompilerParams(
            dimension_semantics=("arbitrary",)),
    )(lab_r, ms_r, emb_r, pe_body, cls_token, pos0, pad_token)

    return batch, labout.reshape(B, NP), attn
